# baseline (speedup 1.0000x reference)
import functools

import jax
import jax.numpy as jnp
from jax import lax
from jax.experimental import pallas as pl
from jax.experimental.pallas import tpu as pltpu, tpu_sc as plsc

_BATCH = 16384
_TABLE = 1000
_TABLE_PAD = 1024
_LANES = 16


def _make_kernel():
    info = plsc.get_sparse_core_info()
    nc, ns = 1, info.num_subcores
    nw = nc * ns
    b_per_w = _BATCH // nw

    mesh = plsc.VectorSubcoreMesh(
        core_axis_name="c", subcore_axis_name="s", num_cores=nc
    )

    n_chunks = 4
    cs = b_per_w // n_chunks

    @functools.partial(
        pl.kernel,
        out_type=jax.ShapeDtypeStruct((_BATCH,), jnp.float32),
        mesh=mesh,
        scratch_types=[
            pltpu.VMEM((_TABLE,), jnp.float32),
            pltpu.VMEM((b_per_w,), jnp.int32),
            pltpu.VMEM((b_per_w,), jnp.float32),
            pltpu.SemaphoreType.DMA,
            [pltpu.SemaphoreType.DMA] * n_chunks,
            pltpu.SemaphoreType.DMA,
        ],
        compiler_params=pltpu.CompilerParams(needs_layout_passes=False),
    )
    def beta_gather(
        idx_hbm, betas_hbm, out_hbm, table_v, idx_v, out_v, sem_t, sems_i, sem_o
    ):
        wid = lax.axis_index("s") * nc + lax.axis_index("c")
        base = wid * b_per_w
        cp_t = pltpu.async_copy(betas_hbm, table_v, sem_t)
        cps_i = [
            pltpu.async_copy(
                idx_hbm.at[pl.ds(base + c * cs, cs)],
                idx_v.at[pl.ds(c * cs, cs)],
                sems_i[c],
            )
            for c in range(n_chunks)
        ]
        cp_t.wait()
        cps_o = []
        for c in range(n_chunks):
            cps_i[c].wait()
            for i in range(c * cs // _LANES, (c + 1) * cs // _LANES):
                ids = idx_v[pl.ds(i * _LANES, _LANES)]
                out_v[pl.ds(i * _LANES, _LANES)] = plsc.load_gather(table_v, [ids])
            cps_o.append(
                pltpu.async_copy(
                    out_v.at[pl.ds(c * cs, cs)],
                    out_hbm.at[pl.ds(base + c * cs, cs)],
                    sem_o,
                )
            )
        for cp in cps_o:
            cp.wait()

    return beta_gather


_beta_gather = _make_kernel()


@jax.jit
def kernel(num_steps, betas):
    out = _beta_gather(num_steps, betas)
    return out.reshape((_BATCH, 1))


# speedup vs baseline: 1.0018x; 1.0018x over previous
import functools

import jax
import jax.numpy as jnp
from jax import lax
from jax.experimental import pallas as pl
from jax.experimental.pallas import tpu as pltpu, tpu_sc as plsc

_BATCH = 16384
_TABLE = 1000
_LANES = 16


def _make_kernel():
    info = plsc.get_sparse_core_info()
    nc, ns = 1, info.num_subcores
    nw = nc * ns
    b_per_w = _BATCH // nw

    mesh = plsc.VectorSubcoreMesh(
        core_axis_name="c", subcore_axis_name="s", num_cores=nc
    )

    n_chunks = 4
    cs = b_per_w // n_chunks

    @functools.partial(
        pl.kernel,
        out_type=jax.ShapeDtypeStruct((_BATCH,), jnp.float32),
        mesh=mesh,
        scratch_types=[
            pltpu.VMEM((_TABLE,), jnp.float32),
            pltpu.VMEM((b_per_w,), jnp.int32),
            pltpu.VMEM((b_per_w,), jnp.float32),
            pltpu.SemaphoreType.DMA,
            [pltpu.SemaphoreType.DMA] * n_chunks,
            pltpu.SemaphoreType.DMA,
        ],
        compiler_params=pltpu.CompilerParams(needs_layout_passes=False),
    )
    def beta_gather(
        idx_hbm, betas_hbm, out_hbm, table_v, idx_v, out_v, sem_t, sems_i, sem_o
    ):
        wid = lax.axis_index("s") * nc + lax.axis_index("c")
        base = wid * b_per_w
        cp_t = pltpu.async_copy(betas_hbm, table_v, sem_t)
        cps_i = [
            pltpu.async_copy(
                idx_hbm.at[pl.ds(base + c * cs, cs)],
                idx_v.at[pl.ds(c * cs, cs)],
                sems_i[c],
            )
            for c in range(n_chunks)
        ]
        cp_t.wait()
        cps_o = []
        for c in range(n_chunks):
            cps_i[c].wait()
            for i in range(c * cs // _LANES, (c + 1) * cs // _LANES):
                ids = idx_v[pl.ds(i * _LANES, _LANES)]
                out_v[pl.ds(i * _LANES, _LANES)] = plsc.load_gather(table_v, [ids])
            cps_o.append(
                pltpu.async_copy(
                    out_v.at[pl.ds(c * cs, cs)],
                    out_hbm.at[pl.ds(base + c * cs, cs)],
                    sem_o,
                )
            )
        for cp in cps_o:
            cp.wait()

    return beta_gather


_beta_gather = _make_kernel()


@jax.jit
def kernel(num_steps, betas):
    out = _beta_gather(num_steps, betas)
    return out.reshape((_BATCH, 1))
